# Initial kernel scaffold; baseline (speedup 1.0000x reference)
import jax
import jax.numpy as jnp
from jax.experimental import pallas as pl


def kernel(x, split_coefs, leaves_feat_ids, leaves_coefs):
    raise NotImplementedError("write your pallas kernel here")


# speedup vs baseline: 1.5113x; 1.5113x over previous
import functools

import jax
import jax.numpy as jnp
import numpy as np
from jax.experimental import pallas as pl
from jax.experimental.pallas import tpu as pltpu

_B = 2048
_IN = 512
_OUT = 32
_T = 32
_DEPTH = 6
_S = 31
_L = 32
_F = 16
_TL = _T * _L
_K = _TL * _F

_BT = 256
_NLC = 128
_CK = _NLC * _F
_NCHUNK = _TL // _NLC


def _smooth_step(t):
    tc = jnp.clip(t, -0.5, 0.5)
    return tc * (1.5 - 2.0 * tc * tc) + 0.5


def _tree_kernel(xT_ref, slopes_ref, bias_ref, ids_ref, w2f_ref, wb_ref,
                 out_ref):
    xT = xT_ref[...]
    t = jax.lax.dot_general(
        slopes_ref[...], xT, (((1,), (0,)), ((), ())),
        preferred_element_type=jnp.float32)
    s = _smooth_step(t + bias_ref[...])
    s3 = s.reshape(_T, _S, _BT)
    aT = None
    for d in range(_DEPTH - 1):
        nb, ne = 2 ** d - 1, 2 ** (d + 1) - 1
        lvl = s3[:, nb:ne, :].reshape(_T, ne - nb, 1, _BT)
        rep = jnp.broadcast_to(lvl, (_T, ne - nb, _L // (ne - nb), _BT))
        rep = rep.reshape(_T, _L, _BT)
        lidx = jax.lax.broadcasted_iota(jnp.int32, (1, _L, 1), 1)
        bit = ((lidx >> (_DEPTH - 2 - d)) & 1).astype(jnp.float32)
        f = (2.0 * bit - 1.0) * rep + (1.0 - bit)
        aT = f if aT is None else aT * f
    aT = aT.reshape(_TL, _BT)

    xh = xT.astype(jnp.bfloat16)
    xl = (xT - xh.astype(jnp.float32)).astype(jnp.bfloat16)
    acc = jnp.zeros((_BT, _OUT), jnp.float32)
    for c in range(_NCHUNK):
        ids_col = ids_ref[c].reshape(_CK, 1)
        iota = jax.lax.broadcasted_iota(jnp.int32, (_CK, _IN), 1)
        poh = (iota == ids_col).astype(jnp.bfloat16)
        xf = jax.lax.dot_general(
            poh, xh, (((1,), (0,)), ((), ())),
            preferred_element_type=jnp.float32)
        xf = xf + jax.lax.dot_general(
            poh, xl, (((1,), (0,)), ((), ())),
            preferred_element_type=jnp.float32)
        a_c = aT[c * _NLC:(c + 1) * _NLC, :].reshape(_NLC, 1, _BT)
        a_rep = jnp.broadcast_to(a_c, (_NLC, _F, _BT)).reshape(_CK, _BT)
        y = a_rep * xf
        acc = acc + jax.lax.dot_general(
            y, w2f_ref[c * _CK:(c + 1) * _CK, :], (((0,), (0,)), ((), ())),
            preferred_element_type=jnp.float32)

    out_ref[...] = acc + jax.lax.dot_general(
        aT, wb_ref[...], (((0,), (0,)), ((), ())),
        preferred_element_type=jnp.float32)


@jax.jit
def kernel(x, split_coefs, leaves_feat_ids, leaves_coefs):
    xT = x.T
    slopes = split_coefs[:, :, :-1].reshape(_T * _S, _IN)
    bias = split_coefs[:, :, -1].reshape(_T * _S, 1)
    ids = leaves_feat_ids.astype(jnp.int32).reshape(_NCHUNK, _CK)
    w2f = jnp.transpose(leaves_coefs[:, :, :, :_F], (0, 1, 3, 2))
    w2f = w2f.reshape(_K, _OUT)
    wb = leaves_coefs[:, :, :, _F].reshape(_TL, _OUT)

    grid = (_B // _BT,)
    return pl.pallas_call(
        _tree_kernel,
        grid=grid,
        in_specs=[
            pl.BlockSpec((_IN, _BT), lambda i: (0, i)),
            pl.BlockSpec((_T * _S, _IN), lambda i: (0, 0)),
            pl.BlockSpec((_T * _S, 1), lambda i: (0, 0)),
            pl.BlockSpec((_NCHUNK, _CK), lambda i: (0, 0)),
            pl.BlockSpec((_K, _OUT), lambda i: (0, 0)),
            pl.BlockSpec((_TL, _OUT), lambda i: (0, 0)),
        ],
        out_specs=pl.BlockSpec((_BT, _OUT), lambda i: (i, 0)),
        out_shape=jax.ShapeDtypeStruct((_B, _OUT), jnp.float32),
    )(xT, slopes, bias, ids, w2f, wb)
